# baseline (speedup 1.0000x reference)
import functools

import jax
import jax.numpy as jnp
from jax import lax
from jax.experimental import pallas as pl
from jax.experimental.pallas import tpu as pltpu
from jax.experimental.pallas import tpu_sc as plsc

_B, _P, _E = 32, 196, 512
_V, _EMB, _ATT, _L = 10000, 256, 256, 512
_S = 50
_T = _S - 1
_TP = 56
_NW = 32
_IPW = 56
_NIDX = _NW * _IPW
_CH = 8

_F32 = jnp.float32
_BF = jnp.bfloat16
_HI = jax.lax.Precision.HIGHEST


def _sort_body(enc_ref, seqs_ref, lc_ref, lens_ref,
               encs_ref, seqs_out_ref, iter_out_ref, sidx_out_ref, inv_ref):
    lvals = [lens_ref[i, 0] for i in range(_B)]
    for i in range(_B):
        cnt = jnp.int32(0)
        for j in range(_B):
            if j < i:
                cnt = cnt + (lvals[j] >= lvals[i]).astype(jnp.int32)
            elif j > i:
                cnt = cnt + (lvals[j] > lvals[i]).astype(jnp.int32)
        inv_ref[cnt, 0] = jnp.int32(i)

    for r in range(_B):
        idx = inv_ref[r, 0]
        encs_ref[pl.ds(r, 1)] = enc_ref[pl.ds(idx, 1)]
        seqs_out_ref[pl.ds(r, 1)] = seqs_ref[pl.ds(idx, 1)]
        iter_out_ref[pl.ds(r, 1)] = lc_ref[pl.ds(idx, 1)] - 1
        sidx_out_ref[pl.ds(r, 1)] = jnp.full((1, 1), idx, jnp.int32)


def _sc_gather_body(table_hbm, idx_hbm, out_hbm, idx_v, rows_v, sem):
    wid = lax.axis_index("s") * 2 + lax.axis_index("c")
    base = wid * _IPW
    pltpu.sync_copy(idx_hbm.at[pl.ds(base, _IPW)], idx_v)
    pltpu.async_copy(table_hbm.at[idx_v], rows_v, sem).wait()
    pltpu.sync_copy(rows_v, out_hbm.at[pl.ds(base, _IPW)])


@functools.cache
def _sc_gather():
    return pl.kernel(
        _sc_gather_body,
        out_type=jax.ShapeDtypeStruct((_NIDX, _EMB), _F32),
        mesh=plsc.VectorSubcoreMesh(core_axis_name="c", subcore_axis_name="s"),
        scratch_types=[
            pltpu.VMEM((_IPW,), jnp.int32),
            pltpu.VMEM((_IPW, _EMB), _F32),
            pltpu.SemaphoreType.DMA,
        ],
    )


def _embed_gather(emb_table, idx_flat):
    return _sc_gather()(emb_table, idx_flat)


def _recur_body(enc_ref, emb_ref, iterv_ref, iters_ref,
                wea_ref, bea_ref, wc2_ref, bc2_ref, wf_ref, bf_ref,
                winit_ref, binit_ref, wk_ref, wrec_ref, bl_ref,
                c_out_ref, al_out_ref,
                a1_ref, ct_ref, at_ref, lg_ref, awe_ref):
    menc = jnp.concatenate(
        [jnp.sum(enc_ref[i * _CH:(i + 1) * _CH], axis=1)
         for i in range(_B // _CH)], axis=0) * (1.0 / _P)
    init = jnp.dot(menc, winit_ref[...], preferred_element_type=_F32) + binit_ref[...]
    h0 = init[:, :_L]
    c0 = init[:, _L:]
    for b in range(_B):
        a1_ref[b] = (
            jnp.dot(enc_ref[b], wea_ref[...], preferred_element_type=_F32)
            + bea_ref[...]
        )

    wc2 = wc2_ref[...]
    bc2 = bc2_ref[...]
    wk = wk_ref[...]
    wrec = wrec_ref[...]
    bl = bl_ref[...]
    wf = wf_ref[...]
    bf = bf_ref[...]
    iterv = iterv_ref[...]

    def step(t, hc):
        h, c = hc
        cproj = jnp.dot(c, wc2, preferred_element_type=_F32) + bc2
        a2 = cproj[:, :_ATT]
        beta = jax.nn.sigmoid(cproj[:, _ATT:])
        for i in range(_B // _CH):
            lo = i * _CH

            @pl.when(t < iters_ref[lo, 0])
            def _():
                rc = jax.nn.relu(a1_ref[lo:lo + _CH]
                                 + a2[lo:lo + _CH][:, None, :])
                lg_ref[lo:lo + _CH] = jnp.sum(rc * wf[None, :, :], axis=-1)

        logits = lg_ref[...] + bf
        mm = jnp.max(logits, axis=-1, keepdims=True)
        ee = jnp.exp(logits - mm)
        alpha = ee * (1.0 / jnp.sum(ee, axis=-1, keepdims=True))
        for i in range(_B // _CH):
            lo = i * _CH

            @pl.when(t < iters_ref[lo, 0])
            def _():
                awe_ref[lo:lo + _CH] = jnp.sum(
                    enc_ref[lo:lo + _CH] * alpha[lo:lo + _CH][:, :, None],
                    axis=1)

        awe = awe_ref[...] * beta
        emb_t = emb_ref[t]
        x = jnp.concatenate([emb_t, awe], axis=1)
        z = (jnp.dot(x, wk, preferred_element_type=_F32)
             + jnp.dot(h, wrec, preferred_element_type=_F32) + bl)
        i_g = jax.nn.sigmoid(z[:, :_L])
        f_g = jax.nn.sigmoid(z[:, _L:2 * _L])
        c_new = f_g * c + i_g * jnp.tanh(z[:, 2 * _L:3 * _L])
        o_g = jax.nn.sigmoid(z[:, 3 * _L:])
        h_new = o_g * jnp.tanh(c_new)
        mask = iterv > t
        ct_ref[t] = c_new
        at_ref[t] = jnp.where(mask, alpha, 0.0)
        return (jnp.where(mask, h_new, h), jnp.where(mask, c_new, c))

    lax.fori_loop(0, _T, step, (h0, c0))

    for t in range(_T):
        c_out_ref[:, t, :] = ct_ref[t]
        al_out_ref[:, t, :] = at_ref[t]
    c_out_ref[:, _T:_TP, :] = jnp.zeros((_B, _TP - _T, _L), _F32)


def _vocab_body(c_ref, w_ref, b_ref, iterv_ref, iters_ref, out_ref):
    bb = c_ref.shape[0]
    cb = c_ref[...].reshape(bb * _TP, _L)
    logits = jnp.dot(cb, w_ref[...], preferred_element_type=_F32) + b_ref[...]
    l3 = logits.reshape(bb, _TP, _V)[:, :_T, :]
    m = jnp.max(l3, axis=-1, keepdims=True)
    e = jnp.exp(l3 - m)
    p3 = e * (1.0 / jnp.sum(e, axis=-1, keepdims=True))
    tt = lax.broadcasted_iota(jnp.int32, (bb, _T, 1), 1)
    mask = tt < iterv_ref[0][:, :, None]
    out_ref[...] = jnp.where(mask, p3, 0.0)


def kernel(encoder_output, sequences, sequence_lengths, emb_table,
           W_enc_att, b_enc_att, W_gen_att, b_gen_att, W_full, b_full,
           W_init_m, b_init_m, W_init_c, b_init_c, W_beta, b_beta,
           W_kernel, W_rec, b_lstm, W_out, b_out):
    seqs32 = sequences.astype(jnp.int32)
    lens32 = sequence_lengths.astype(jnp.int32)
    lc = lens32.reshape(_B, 1)
    lr = lens32.reshape(1, _B)

    enc_s, seqs_sorted, iter2d, sidx2d = pl.pallas_call(
        _sort_body,
        out_shape=[
            jax.ShapeDtypeStruct((_B, _P, _E), _F32),
            jax.ShapeDtypeStruct((_B, _S), jnp.int32),
            jax.ShapeDtypeStruct((_B, 1), jnp.int32),
            jax.ShapeDtypeStruct((_B, 1), jnp.int32),
        ],
        in_specs=[
            pl.BlockSpec((_B, _P, _E), lambda: (0, 0, 0)),
            pl.BlockSpec((_B, _S), lambda: (0, 0)),
            pl.BlockSpec((_B, 1), lambda: (0, 0)),
            pl.BlockSpec(memory_space=pltpu.SMEM),
        ],
        scratch_shapes=[pltpu.SMEM((_B, 1), jnp.int32)],
        compiler_params=pltpu.CompilerParams(
            vmem_limit_bytes=100 * 1024 * 1024),
    )(encoder_output, seqs32, lc, lc)

    idx_t = jnp.transpose(seqs_sorted).reshape(-1)
    idx_flat = jnp.concatenate(
        [idx_t, jnp.zeros((_NIDX - _B * _S,), jnp.int32)])
    emb3 = _embed_gather(emb_table, idx_flat).reshape(_NIDX // _B, _B, _EMB)

    wc2 = jnp.concatenate([W_gen_att, W_beta], axis=1)
    bc2 = jnp.concatenate([b_gen_att, b_beta]).reshape(1, _ATT + _E)
    winit = jnp.concatenate([W_init_m, W_init_c], axis=1)
    binit = jnp.concatenate([b_init_m, b_init_c]).reshape(1, 2 * _L)
    bl = b_lstm.reshape(1, 4 * _L)
    wf = W_full.reshape(1, _ATT)
    bf = b_full.reshape(1, 1)
    bea = b_enc_att.reshape(1, _ATT)

    c_pad, alphas = pl.pallas_call(
        _recur_body,
        out_shape=[
            jax.ShapeDtypeStruct((_B, _TP, _L), _F32),
            jax.ShapeDtypeStruct((_B, _T, _P), _F32),
        ],
        in_specs=[
            pl.BlockSpec((_B, _P, _E), lambda: (0, 0, 0)),
            pl.BlockSpec((_NIDX // _B, _B, _EMB), lambda: (0, 0, 0)),
            pl.BlockSpec((_B, 1), lambda: (0, 0)),
            pl.BlockSpec(memory_space=pltpu.SMEM),
            pl.BlockSpec((_E, _ATT), lambda: (0, 0)),
            pl.BlockSpec((1, _ATT), lambda: (0, 0)),
            pl.BlockSpec((_L, _ATT + _E), lambda: (0, 0)),
            pl.BlockSpec((1, _ATT + _E), lambda: (0, 0)),
            pl.BlockSpec((1, _ATT), lambda: (0, 0)),
            pl.BlockSpec((1, 1), lambda: (0, 0)),
            pl.BlockSpec((_E, 2 * _L), lambda: (0, 0)),
            pl.BlockSpec((1, 2 * _L), lambda: (0, 0)),
            pl.BlockSpec((_EMB + _E, 4 * _L), lambda: (0, 0)),
            pl.BlockSpec((_L, 4 * _L), lambda: (0, 0)),
            pl.BlockSpec((1, 4 * _L), lambda: (0, 0)),
        ],
        scratch_shapes=[
            pltpu.VMEM((_B, _P, _ATT), _F32),
            pltpu.VMEM((_T, _B, _L), _F32),
            pltpu.VMEM((_T, _B, _P), _F32),
            pltpu.VMEM((_B, _P), _F32),
            pltpu.VMEM((_B, _E), _F32),
        ],
        compiler_params=pltpu.CompilerParams(
            vmem_limit_bytes=100 * 1024 * 1024),
    )(enc_s, emb3, iter2d, iter2d,
      W_enc_att, bea, wc2, bc2, wf, bf, winit, binit, W_kernel, W_rec, bl)

    bb = 4
    preds = pl.pallas_call(
        _vocab_body,
        grid=(_B // bb,),
        in_specs=[
            pl.BlockSpec((bb, _TP, _L), lambda i: (i, 0, 0)),
            pl.BlockSpec((_L, _V), lambda i: (0, 0)),
            pl.BlockSpec((1, _V), lambda i: (0, 0)),
            pl.BlockSpec((1, bb, 1), lambda i: (i, 0, 0)),
            pl.BlockSpec(memory_space=pltpu.SMEM),
        ],
        out_specs=pl.BlockSpec((bb, _T, _V), lambda i: (i, 0, 0)),
        out_shape=jax.ShapeDtypeStruct((_B, _T, _V), _F32),
        compiler_params=pltpu.CompilerParams(
            vmem_limit_bytes=100 * 1024 * 1024),
    )(c_pad, W_out, b_out.reshape(1, _V), iter2d.reshape(_B // bb, bb, 1),
      iter2d)

    return (preds, alphas, seqs_sorted, iter2d.reshape(_B),
            sidx2d.reshape(_B))


# speedup vs baseline: 1.0072x; 1.0072x over previous
import functools

import jax
import jax.numpy as jnp
from jax import lax
from jax.experimental import pallas as pl
from jax.experimental.pallas import tpu as pltpu
from jax.experimental.pallas import tpu_sc as plsc

_B, _P, _E = 32, 196, 512
_V, _EMB, _ATT, _L = 10000, 256, 256, 512
_S = 50
_T = _S - 1
_TP = 56
_NW = 32
_IPW = 56
_NIDX = _NW * _IPW
_CH = 8

_F32 = jnp.float32
_BF = jnp.bfloat16
_HI = jax.lax.Precision.HIGHEST


def _sort_body(enc_ref, seqs_ref, lc_ref, lens_ref,
               encs_ref, seqs_out_ref, iter_out_ref, sidx_out_ref, inv_ref):
    lvals = [lens_ref[i, 0] for i in range(_B)]
    for i in range(_B):
        cnt = jnp.int32(0)
        for j in range(_B):
            if j < i:
                cnt = cnt + (lvals[j] >= lvals[i]).astype(jnp.int32)
            elif j > i:
                cnt = cnt + (lvals[j] > lvals[i]).astype(jnp.int32)
        inv_ref[cnt, 0] = jnp.int32(i)

    for r in range(_B):
        idx = inv_ref[r, 0]
        encs_ref[pl.ds(r, 1)] = enc_ref[pl.ds(idx, 1)]
        seqs_out_ref[pl.ds(r, 1)] = seqs_ref[pl.ds(idx, 1)]
        iter_out_ref[pl.ds(r, 1)] = lc_ref[pl.ds(idx, 1)] - 1
        sidx_out_ref[pl.ds(r, 1)] = jnp.full((1, 1), idx, jnp.int32)


def _sc_gather_body(table_hbm, idx_hbm, out_hbm, idx_v, rows_v, sem):
    wid = lax.axis_index("s") * 2 + lax.axis_index("c")
    base = wid * _IPW
    pltpu.sync_copy(idx_hbm.at[pl.ds(base, _IPW)], idx_v)
    pltpu.async_copy(table_hbm.at[idx_v], rows_v, sem).wait()
    pltpu.sync_copy(rows_v, out_hbm.at[pl.ds(base, _IPW)])


@functools.cache
def _sc_gather():
    return pl.kernel(
        _sc_gather_body,
        out_type=jax.ShapeDtypeStruct((_NIDX, _EMB), _F32),
        mesh=plsc.VectorSubcoreMesh(core_axis_name="c", subcore_axis_name="s"),
        scratch_types=[
            pltpu.VMEM((_IPW,), jnp.int32),
            pltpu.VMEM((_IPW, _EMB), _F32),
            pltpu.SemaphoreType.DMA,
        ],
    )


def _embed_gather(emb_table, idx_flat):
    return _sc_gather()(emb_table, idx_flat)


def _recur_body(enc_ref, emb_ref, iterv_ref, iters_ref,
                wea_ref, bea_ref, wc2_ref, bc2_ref, wf_ref, bf_ref,
                winit_ref, binit_ref, wk_ref, wrec_ref, bl_ref,
                c_out_ref, al_out_ref,
                a1_ref, ct_ref, at_ref, lg_ref, awe_ref):
    menc = jnp.concatenate(
        [jnp.sum(enc_ref[i * _CH:(i + 1) * _CH], axis=1)
         for i in range(_B // _CH)], axis=0) * (1.0 / _P)
    init = jnp.dot(menc, winit_ref[...], preferred_element_type=_F32) + binit_ref[...]
    h0 = init[:, :_L]
    c0 = init[:, _L:]
    for b in range(_B):
        a1_ref[b] = (
            jnp.dot(enc_ref[b], wea_ref[...], preferred_element_type=_F32)
            + bea_ref[...]
        )

    wc2 = wc2_ref[...]
    bc2 = bc2_ref[...]
    wk = wk_ref[...]
    wrec = wrec_ref[...]
    bl = bl_ref[...]
    wf = wf_ref[...]
    bf = bf_ref[...]
    iterv = iterv_ref[...]

    def step(t, hc):
        h, c = hc
        cproj = jnp.dot(c, wc2, preferred_element_type=_F32) + bc2
        a2 = cproj[:, :_ATT]
        beta = jax.nn.sigmoid(cproj[:, _ATT:])
        for i in range(_B // _CH):
            lo = i * _CH

            @pl.when(t < iters_ref[lo, 0])
            def _():
                rc = jax.nn.relu(a1_ref[lo:lo + _CH]
                                 + a2[lo:lo + _CH][:, None, :])
                lg_ref[lo:lo + _CH] = jnp.sum(rc * wf[None, :, :], axis=-1)

        logits = lg_ref[...] + bf
        mm = jnp.max(logits, axis=-1, keepdims=True)
        ee = jnp.exp(logits - mm)
        alpha = ee * (1.0 / jnp.sum(ee, axis=-1, keepdims=True))
        for i in range(_B // _CH):
            lo = i * _CH

            @pl.when(t < iters_ref[lo, 0])
            def _():
                awe_ref[lo:lo + _CH] = jnp.sum(
                    enc_ref[lo:lo + _CH] * alpha[lo:lo + _CH][:, :, None],
                    axis=1)

        awe = awe_ref[...] * beta
        emb_t = emb_ref[t]
        x = jnp.concatenate([emb_t, awe], axis=1)
        z = (jnp.dot(x, wk, preferred_element_type=_F32)
             + jnp.dot(h, wrec, preferred_element_type=_F32) + bl)
        i_g = jax.nn.sigmoid(z[:, :_L])
        f_g = jax.nn.sigmoid(z[:, _L:2 * _L])
        c_new = f_g * c + i_g * jnp.tanh(z[:, 2 * _L:3 * _L])
        o_g = jax.nn.sigmoid(z[:, 3 * _L:])
        h_new = o_g * jnp.tanh(c_new)
        mask = iterv > t
        ct_ref[t] = c_new
        at_ref[t] = jnp.where(mask, alpha, 0.0)
        return (jnp.where(mask, h_new, h), jnp.where(mask, c_new, c))

    lax.fori_loop(0, _T, step, (h0, c0))

    for t in range(_T):
        c_out_ref[:, t, :] = ct_ref[t]
        al_out_ref[:, t, :] = at_ref[t]
    c_out_ref[:, _T:_TP, :] = jnp.zeros((_B, _TP - _T, _L), _F32)


def _vocab_body(c_ref, w_ref, b_ref, iterv_ref, iters_ref, out_ref):
    bb = c_ref.shape[0]
    cb = c_ref[...].reshape(bb * _TP, _L)
    logits = jnp.dot(cb, w_ref[...], preferred_element_type=_F32) + b_ref[...]
    m = jnp.max(logits, axis=-1, keepdims=True)
    e = jnp.exp(logits - m)
    probs = e * (1.0 / jnp.sum(e, axis=-1, keepdims=True))
    p3 = probs.reshape(bb, _TP, _V)
    tt = lax.broadcasted_iota(jnp.int32, (bb, _TP, 1), 1)
    mask = tt < iterv_ref[0][:, :, None]
    p3 = jnp.where(mask, p3, 0.0)
    out_ref[...] = p3[:, :_T, :]


def kernel(encoder_output, sequences, sequence_lengths, emb_table,
           W_enc_att, b_enc_att, W_gen_att, b_gen_att, W_full, b_full,
           W_init_m, b_init_m, W_init_c, b_init_c, W_beta, b_beta,
           W_kernel, W_rec, b_lstm, W_out, b_out):
    seqs32 = sequences.astype(jnp.int32)
    lens32 = sequence_lengths.astype(jnp.int32)
    lc = lens32.reshape(_B, 1)
    lr = lens32.reshape(1, _B)

    enc_s, seqs_sorted, iter2d, sidx2d = pl.pallas_call(
        _sort_body,
        out_shape=[
            jax.ShapeDtypeStruct((_B, _P, _E), _F32),
            jax.ShapeDtypeStruct((_B, _S), jnp.int32),
            jax.ShapeDtypeStruct((_B, 1), jnp.int32),
            jax.ShapeDtypeStruct((_B, 1), jnp.int32),
        ],
        in_specs=[
            pl.BlockSpec((_B, _P, _E), lambda: (0, 0, 0)),
            pl.BlockSpec((_B, _S), lambda: (0, 0)),
            pl.BlockSpec((_B, 1), lambda: (0, 0)),
            pl.BlockSpec(memory_space=pltpu.SMEM),
        ],
        scratch_shapes=[pltpu.SMEM((_B, 1), jnp.int32)],
        compiler_params=pltpu.CompilerParams(
            vmem_limit_bytes=100 * 1024 * 1024),
    )(encoder_output, seqs32, lc, lc)

    idx_t = jnp.transpose(seqs_sorted).reshape(-1)
    idx_flat = jnp.concatenate(
        [idx_t, jnp.zeros((_NIDX - _B * _S,), jnp.int32)])
    emb3 = _embed_gather(emb_table, idx_flat).reshape(_NIDX // _B, _B, _EMB)

    wc2 = jnp.concatenate([W_gen_att, W_beta], axis=1)
    bc2 = jnp.concatenate([b_gen_att, b_beta]).reshape(1, _ATT + _E)
    winit = jnp.concatenate([W_init_m, W_init_c], axis=1)
    binit = jnp.concatenate([b_init_m, b_init_c]).reshape(1, 2 * _L)
    bl = b_lstm.reshape(1, 4 * _L)
    wf = W_full.reshape(1, _ATT)
    bf = b_full.reshape(1, 1)
    bea = b_enc_att.reshape(1, _ATT)

    c_pad, alphas = pl.pallas_call(
        _recur_body,
        out_shape=[
            jax.ShapeDtypeStruct((_B, _TP, _L), _F32),
            jax.ShapeDtypeStruct((_B, _T, _P), _F32),
        ],
        in_specs=[
            pl.BlockSpec((_B, _P, _E), lambda: (0, 0, 0)),
            pl.BlockSpec((_NIDX // _B, _B, _EMB), lambda: (0, 0, 0)),
            pl.BlockSpec((_B, 1), lambda: (0, 0)),
            pl.BlockSpec(memory_space=pltpu.SMEM),
            pl.BlockSpec((_E, _ATT), lambda: (0, 0)),
            pl.BlockSpec((1, _ATT), lambda: (0, 0)),
            pl.BlockSpec((_L, _ATT + _E), lambda: (0, 0)),
            pl.BlockSpec((1, _ATT + _E), lambda: (0, 0)),
            pl.BlockSpec((1, _ATT), lambda: (0, 0)),
            pl.BlockSpec((1, 1), lambda: (0, 0)),
            pl.BlockSpec((_E, 2 * _L), lambda: (0, 0)),
            pl.BlockSpec((1, 2 * _L), lambda: (0, 0)),
            pl.BlockSpec((_EMB + _E, 4 * _L), lambda: (0, 0)),
            pl.BlockSpec((_L, 4 * _L), lambda: (0, 0)),
            pl.BlockSpec((1, 4 * _L), lambda: (0, 0)),
        ],
        scratch_shapes=[
            pltpu.VMEM((_B, _P, _ATT), _F32),
            pltpu.VMEM((_T, _B, _L), _F32),
            pltpu.VMEM((_T, _B, _P), _F32),
            pltpu.VMEM((_B, _P), _F32),
            pltpu.VMEM((_B, _E), _F32),
        ],
        compiler_params=pltpu.CompilerParams(
            vmem_limit_bytes=100 * 1024 * 1024),
    )(enc_s, emb3, iter2d, iter2d,
      W_enc_att, bea, wc2, bc2, wf, bf, winit, binit, W_kernel, W_rec, bl)

    bb = 4
    preds = pl.pallas_call(
        _vocab_body,
        grid=(_B // bb,),
        in_specs=[
            pl.BlockSpec((bb, _TP, _L), lambda i: (i, 0, 0)),
            pl.BlockSpec((_L, _V), lambda i: (0, 0)),
            pl.BlockSpec((1, _V), lambda i: (0, 0)),
            pl.BlockSpec((1, bb, 1), lambda i: (i, 0, 0)),
            pl.BlockSpec(memory_space=pltpu.SMEM),
        ],
        out_specs=pl.BlockSpec((bb, _T, _V), lambda i: (i, 0, 0)),
        out_shape=jax.ShapeDtypeStruct((_B, _T, _V), _F32),
        compiler_params=pltpu.CompilerParams(
            vmem_limit_bytes=100 * 1024 * 1024),
    )(c_pad, W_out, b_out.reshape(1, _V), iter2d.reshape(_B // bb, bb, 1),
      iter2d)

    return (preds, alphas, seqs_sorted, iter2d.reshape(_B),
            sidx2d.reshape(_B))
